# baseline (speedup 1.0000x reference)
import functools

import jax
import jax.numpy as jnp
from jax import lax
from jax.experimental import pallas as pl
from jax.experimental.pallas import tpu as pltpu
from jax.experimental.pallas import tpu_sc as plsc

_L = 16
_NC = 2
_NS = 16
_D = 32
_V = 1000000
_B = 16384
_TCW = 128
_CW = 512
_IP = 4096
_E = 2048
_NFULL_TC = _V // _TCW
_TAIL = _V - _NFULL_TC * _TCW
_TC_PER_SC = _NFULL_TC // _NC
_TC_SHARE = _TC_PER_SC // _NS
_TC_REM = _TC_PER_SC % _NS


def _fast_rsqrt(x):
    i = plsc.bitcast(x, jnp.int32)
    i = jnp.int32(0x5F3759DF) - (i >> 1)
    y = plsc.bitcast(i, jnp.float32)
    xh = x * jnp.float32(0.5)
    for _ in range(3):
        y = y * (jnp.float32(1.5) - xh * y * y)
    return y


def _make_kernel():
    mesh = plsc.VectorSubcoreMesh(
        core_axis_name="c", subcore_axis_name="s", num_cores=_NC,
        num_subcores=_NS,
    )

    @functools.partial(
        pl.kernel,
        out_type=jax.ShapeDtypeStruct((_B + 8, _TCW), jnp.float32),
        mesh=mesh,
        scratch_types=[
            pltpu.VMEM((_IP,), jnp.int32),
            pltpu.VMEM((_E,), jnp.int32),
            pltpu.VMEM((_E,), jnp.int32),
            pltpu.VMEM((2, _D, _CW), jnp.float32),
            pltpu.VMEM((_L, _TCW), jnp.float32),
            pltpu.VMEM((1, _L), jnp.int32),
            pltpu.SemaphoreType.DMA((2,)),
        ],
        compiler_params=pltpu.CompilerParams(
            needs_layout_passes=False, use_tc_tiling_on_sc=True
        ),
    )
    def body(table_t, tail_hbm, idx_hbm, out_p, idxp, ent_lc, ent_pos, chunks,
             staging, posref, sems):
        c = lax.axis_index("c")
        s = lax.axis_index("s")
        lanes = lax.iota(jnp.int32, _L)

        my_tc0 = c * _TC_PER_SC + s * _TC_SHARE + jnp.minimum(s, _TC_REM)
        my_ntc = _TC_SHARE + jnp.where(s < _TC_REM, 1, 0)
        col_lo = my_tc0 * _TCW
        col_n = my_ntc * _TCW
        has_tail = jnp.logical_and(c == _NC - 1, s == _NS - 1)
        tail_cb = _NFULL_TC * _TCW - col_lo

        def filter_pass(skip):
            def piece(p, cnt_vec):
                pltpu.sync_copy(idx_hbm.at[pl.ds(p * _IP, _IP)], idxp)

                def scan(t, cv):
                    v = idxp[pl.ds(t * _L, _L)]
                    pos = p * _IP + t * _L + lanes
                    lc = v - col_lo
                    in_rng = jnp.logical_and(lc >= 0, lc < col_n)
                    in_rng = jnp.logical_or(
                        in_rng,
                        jnp.logical_and(has_tail, v >= _NFULL_TC * _TCW),
                    )
                    mi = in_rng.astype(jnp.int32)
                    slots = cv + plsc.cumsum(mi) - 1 - skip
                    okm = jnp.logical_and(
                        in_rng,
                        jnp.logical_and(slots >= 0, slots < _E),
                    )
                    slots_c = jnp.clip(slots, 0, _E - 1)
                    plsc.store_scatter(ent_lc, [slots_c], lc, mask=okm)
                    plsc.store_scatter(ent_pos, [slots_c], pos, mask=okm)
                    return cv + plsc.all_reduce_population_count(in_rng)

                return lax.fori_loop(0, _IP // _L, scan, cnt_vec)

            cnt_vec = lax.fori_loop(
                0, _B // _IP, piece, jnp.zeros((_L,), jnp.int32)
            )
            return lax.reduce_max(cnt_vec, (0,))

        def process_chunk(buf_sel, cb, width, n_groups, cnt_r):
            def grp(e, carry):
                lc = ent_lc[pl.ds(e * _L, _L)]
                pos = ent_pos[pl.ds(e * _L, _L)]
                valid = (e * _L + lanes) < cnt_r
                lcl = lc - cb
                m2 = jnp.logical_and(
                    valid, jnp.logical_and(lcl >= 0, lcl < width)
                )
                any_hit = lax.reduce_max(m2.astype(jnp.int32), (0,))

                @pl.when(any_hit > 0)
                def _():
                    lcc = jnp.where(m2, lcl, 0)
                    ss = jnp.zeros((_L,), jnp.float32)
                    vals = []
                    for r in range(_D):
                        rv = jnp.full((_L,), r, jnp.int32)
                        x = plsc.load_gather(chunks.at[buf_sel], [rv, lcc])
                        vals.append(x)
                        ss = ss + x * x
                    rinv = _fast_rsqrt(jnp.maximum(ss, jnp.float32(1e-24)))
                    for r in range(_D):
                        rv = jnp.full((_L,), r, jnp.int32)
                        plsc.store_scatter(
                            staging, [lanes, rv], vals[r] * rinv
                        )
                    posref[0, pl.ds(0, _L)] = jnp.where(m2, pos, _B)
                    pltpu.sync_copy(staging, out_p.at[posref.at[0]])

                return carry

            lax.fori_loop(0, n_groups, grp, 0)

        def start_dma(k, b):
            pltpu.async_copy(
                table_t.at[:, pl.ds(col_lo + k * _CW, _CW)],
                chunks.at[b],
                sems.at[b],
            )

        def wait_dma(b):
            pltpu.make_async_copy(
                table_t.at[:, pl.ds(0, _CW)], chunks.at[b], sems.at[b]
            ).wait()

        n_full = my_ntc // 4
        extra_tc = my_ntc - n_full * 4

        def round_body(cnt_r):
            n_groups = (cnt_r + _L - 1) // _L

            @pl.when(n_full > 0)
            def _():
                start_dma(0, 0)

            def outer(t, carry):
                for b in range(2):
                    k = t * 2 + b

                    @pl.when(k < n_full)
                    def _():
                        wait_dma(b)

                        @pl.when(k + 1 < n_full)
                        def _():
                            start_dma(k + 1, 1 - b)

                        process_chunk(b, k * _CW, _CW, n_groups, cnt_r)

                return carry

            lax.fori_loop(0, (_TC_SHARE // 4 + 2) // 2, outer, 0)

            @pl.when(extra_tc > 0)
            def _():
                pltpu.async_copy(
                    table_t.at[:, pl.ds(col_lo + n_full * _CW, _TCW)],
                    chunks.at[0, :, pl.ds(0, _TCW)],
                    sems.at[0],
                )
                pltpu.make_async_copy(
                    table_t.at[:, pl.ds(0, _TCW)],
                    chunks.at[0, :, pl.ds(0, _TCW)],
                    sems.at[0],
                ).wait()
                process_chunk(0, n_full * _CW, _TCW, n_groups, cnt_r)

            @pl.when(has_tail)
            def _():
                pltpu.sync_copy(tail_hbm, chunks.at[0, :, pl.ds(0, _TCW)])
                process_chunk(0, tail_cb, _TAIL, n_groups, cnt_r)

        cnt = filter_pass(0)
        round_body(jnp.minimum(cnt, _E))
        n_rounds = (cnt + _E - 1) // _E

        def extra_round(r, carry):
            filter_pass(r * _E)
            round_body(jnp.minimum(cnt - r * _E, _E))
            return carry

        lax.fori_loop(1, n_rounds, extra_round, 0)

    return body


def kernel(action, table):
    idx = action.astype(jnp.int32)
    table_t = table.T
    tail = jnp.pad(table_t[:, _NFULL_TC * _TCW :], ((0, 0), (0, _TCW - _TAIL)))
    out_p = _make_kernel()(table_t, tail, idx)
    return out_p[:_B, :_D]


# speedup vs baseline: 1.0005x; 1.0005x over previous
import functools

import jax
import jax.numpy as jnp
from jax import lax
from jax.experimental import pallas as pl
from jax.experimental.pallas import tpu as pltpu
from jax.experimental.pallas import tpu_sc as plsc

_L = 16
_NC = 2
_NS = 16
_D = 32
_V = 1000000
_B = 16384
_TCW = 128
_CW = 512
_IP = 4096
_E = 2048
_NFULL_TC = _V // _TCW
_TAIL = _V - _NFULL_TC * _TCW
_TC_PER_SC = _NFULL_TC // _NC
_TC_SHARE = _TC_PER_SC // _NS
_TC_REM = _TC_PER_SC % _NS


def _fast_rsqrt(x):
    i = plsc.bitcast(x, jnp.int32)
    i = jnp.int32(0x5F3759DF) - (i >> 1)
    y = plsc.bitcast(i, jnp.float32)
    xh = x * jnp.float32(0.5)
    for _ in range(3):
        y = y * (jnp.float32(1.5) - xh * y * y)
    return y


def _make_kernel():
    mesh = plsc.VectorSubcoreMesh(
        core_axis_name="c", subcore_axis_name="s", num_cores=_NC,
        num_subcores=_NS,
    )

    @functools.partial(
        pl.kernel,
        out_type=jax.ShapeDtypeStruct((_B + 8, _TCW), jnp.float32),
        mesh=mesh,
        scratch_types=[
            pltpu.VMEM((_IP,), jnp.int32),
            pltpu.VMEM((_E,), jnp.int32),
            pltpu.VMEM((_E,), jnp.int32),
            pltpu.VMEM((2, _D, _CW), jnp.float32),
            pltpu.VMEM((_L, _TCW), jnp.float32),
            pltpu.VMEM((1, _L), jnp.int32),
            pltpu.SemaphoreType.DMA((2,)),
        ],
        compiler_params=pltpu.CompilerParams(
            needs_layout_passes=False, use_tc_tiling_on_sc=True
        ),
    )
    def body(table_t, tail_hbm, idx_hbm, out_p, idxp, ent_lc, ent_pos, chunks,
             staging, posref, sems):
        c = lax.axis_index("c")
        s = lax.axis_index("s")
        lanes = lax.iota(jnp.int32, _L)

        my_tc0 = c * _TC_PER_SC + s * _TC_SHARE + jnp.minimum(s, _TC_REM)
        my_ntc = _TC_SHARE + jnp.where(s < _TC_REM, 1, 0)
        col_lo = my_tc0 * _TCW
        col_n = my_ntc * _TCW
        has_tail = jnp.logical_and(c == _NC - 1, s == _NS - 1)
        tail_cb = _NFULL_TC * _TCW - col_lo

        def filter_pass(skip):
            def piece(p, cnt_vec):
                pltpu.sync_copy(idx_hbm.at[pl.ds(p * _IP, _IP)], idxp)

                def scan(t, cv):
                    v = idxp[pl.ds(t * _L, _L)]
                    pos = p * _IP + t * _L + lanes
                    lc = v - col_lo
                    in_rng = jnp.logical_and(lc >= 0, lc < col_n)
                    in_rng = jnp.logical_or(
                        in_rng,
                        jnp.logical_and(has_tail, v >= _NFULL_TC * _TCW),
                    )
                    mi = in_rng.astype(jnp.int32)
                    slots = cv + plsc.cumsum(mi) - 1 - skip
                    okm = jnp.logical_and(
                        in_rng,
                        jnp.logical_and(slots >= 0, slots < _E),
                    )
                    slots_c = jnp.clip(slots, 0, _E - 1)
                    plsc.store_scatter(ent_lc, [slots_c], lc, mask=okm)
                    plsc.store_scatter(ent_pos, [slots_c], pos, mask=okm)
                    return cv + plsc.all_reduce_population_count(in_rng)

                return lax.fori_loop(0, _IP // _L, scan, cnt_vec)

            cnt_vec = lax.fori_loop(
                0, _B // _IP, piece, jnp.zeros((_L,), jnp.int32)
            )
            return lax.reduce_max(cnt_vec, (0,))

        def process_chunk(buf_sel, cb, width, n_groups, cnt_r):
            def grp(e, carry):
                lc = ent_lc[pl.ds(e * _L, _L)]
                pos = ent_pos[pl.ds(e * _L, _L)]
                valid = (e * _L + lanes) < cnt_r
                lcl = lc - cb
                m2 = jnp.logical_and(
                    valid, jnp.logical_and(lcl >= 0, lcl < width)
                )
                any_hit = lax.reduce_max(m2.astype(jnp.int32), (0,))

                @pl.when(any_hit > 0)
                def _():
                    lcc = jnp.where(m2, lcl, 0)
                    ss = jnp.zeros((_L,), jnp.float32)
                    vals = []
                    for r in range(_D):
                        rv = jnp.full((_L,), r, jnp.int32)
                        x = plsc.load_gather(chunks.at[buf_sel], [rv, lcc])
                        vals.append(x)
                        ss = ss + x * x
                    rinv = _fast_rsqrt(jnp.maximum(ss, jnp.float32(1e-24)))
                    for r in range(_D):
                        rv = jnp.full((_L,), r, jnp.int32)
                        plsc.store_scatter(
                            staging, [lanes, rv], vals[r] * rinv
                        )
                    posref[0, pl.ds(0, _L)] = jnp.where(m2, pos, _B)
                    pltpu.sync_copy(staging, out_p.at[posref.at[0]])

                return carry

            lax.fori_loop(0, n_groups, grp, 0)

        def start_dma(k, b):
            for tr in range(_D // 8):
                pltpu.async_copy(
                    table_t.at[pl.ds(tr * 8, 8), pl.ds(col_lo + k * _CW, _CW)],
                    chunks.at[b, pl.ds(tr * 8, 8)],
                    sems.at[b],
                )

        def wait_dma(b):
            for tr in range(_D // 8):
                pltpu.make_async_copy(
                    table_t.at[pl.ds(tr * 8, 8), pl.ds(0, _CW)],
                    chunks.at[b, pl.ds(tr * 8, 8)],
                    sems.at[b],
                ).wait()

        n_full = my_ntc // 4
        extra_tc = my_ntc - n_full * 4

        def round_body(cnt_r):
            n_groups = (cnt_r + _L - 1) // _L

            @pl.when(n_full > 0)
            def _():
                start_dma(0, 0)

            def outer(t, carry):
                for b in range(2):
                    k = t * 2 + b

                    @pl.when(k < n_full)
                    def _():
                        wait_dma(b)

                        @pl.when(k + 1 < n_full)
                        def _():
                            start_dma(k + 1, 1 - b)

                        process_chunk(b, k * _CW, _CW, n_groups, cnt_r)

                return carry

            lax.fori_loop(0, (_TC_SHARE // 4 + 2) // 2, outer, 0)

            @pl.when(extra_tc > 0)
            def _():
                pltpu.async_copy(
                    table_t.at[:, pl.ds(col_lo + n_full * _CW, _TCW)],
                    chunks.at[0, :, pl.ds(0, _TCW)],
                    sems.at[0],
                )
                pltpu.make_async_copy(
                    table_t.at[:, pl.ds(0, _TCW)],
                    chunks.at[0, :, pl.ds(0, _TCW)],
                    sems.at[0],
                ).wait()
                process_chunk(0, n_full * _CW, _TCW, n_groups, cnt_r)

            @pl.when(has_tail)
            def _():
                pltpu.sync_copy(tail_hbm, chunks.at[0, :, pl.ds(0, _TCW)])
                process_chunk(0, tail_cb, _TAIL, n_groups, cnt_r)

        cnt = filter_pass(0)
        round_body(jnp.minimum(cnt, _E))
        n_rounds = (cnt + _E - 1) // _E

        def extra_round(r, carry):
            filter_pass(r * _E)
            round_body(jnp.minimum(cnt - r * _E, _E))
            return carry

        lax.fori_loop(1, n_rounds, extra_round, 0)

    return body


def kernel(action, table):
    idx = action.astype(jnp.int32)
    table_t = table.T
    tail = jnp.pad(table_t[:, _NFULL_TC * _TCW :], ((0, 0), (0, _TCW - _TAIL)))
    out_p = _make_kernel()(table_t, tail, idx)
    return out_p[:_B, :_D]


# speedup vs baseline: 1.0013x; 1.0008x over previous
import functools

import jax
import jax.numpy as jnp
from jax import lax
from jax.experimental import pallas as pl
from jax.experimental.pallas import tpu as pltpu
from jax.experimental.pallas import tpu_sc as plsc

_L = 16
_NC = 2
_NS = 16
_D = 32
_V = 1000000
_B = 16384
_TCW = 128
_CW = 512
_IP = 4096
_R = 8
_E = 2048
_NFULL_TC = _V // _TCW
_TAIL = _V - _NFULL_TC * _TCW
_TC_PER_SC = _NFULL_TC // _NC
_TC_SHARE = _TC_PER_SC // _NS
_TC_REM = _TC_PER_SC % _NS


def _fast_rsqrt(x):
    i = plsc.bitcast(x, jnp.int32)
    i = jnp.int32(0x5F3759DF) - (i >> 1)
    y = plsc.bitcast(i, jnp.float32)
    xh = x * jnp.float32(0.5)
    for _ in range(3):
        y = y * (jnp.float32(1.5) - xh * y * y)
    return y


def _make_kernel():
    mesh = plsc.VectorSubcoreMesh(
        core_axis_name="c", subcore_axis_name="s", num_cores=_NC,
        num_subcores=_NS,
    )

    @functools.partial(
        pl.kernel,
        out_type=jax.ShapeDtypeStruct((_B + 8, _TCW), jnp.float32),
        mesh=mesh,
        scratch_types=[
            pltpu.VMEM((_IP,), jnp.int32),
            pltpu.VMEM((_E,), jnp.int32),
            pltpu.VMEM((_E,), jnp.int32),
            pltpu.VMEM((2, _D, _CW), jnp.float32),
            pltpu.VMEM((_R, _L, _TCW), jnp.float32),
            pltpu.VMEM((_R, _L), jnp.int32),
            pltpu.VMEM((_L,), jnp.int32),
            pltpu.SemaphoreType.DMA((2,)),
            pltpu.SemaphoreType.DMA,
        ],
        compiler_params=pltpu.CompilerParams(
            needs_layout_passes=False, use_tc_tiling_on_sc=True
        ),
    )
    def body(table_t, tail_hbm, idx_hbm, out_p, idxp, ent_lc, ent_pos, chunks,
             staging, posref, firec, sems, scsem):
        c = lax.axis_index("c")
        s = lax.axis_index("s")
        lanes = lax.iota(jnp.int32, _L)

        my_tc0 = c * _TC_PER_SC + s * _TC_SHARE + jnp.minimum(s, _TC_REM)
        my_ntc = _TC_SHARE + jnp.where(s < _TC_REM, 1, 0)
        col_lo = my_tc0 * _TCW
        col_n = my_ntc * _TCW
        has_tail = jnp.logical_and(c == _NC - 1, s == _NS - 1)
        tail_cb = _NFULL_TC * _TCW - col_lo

        def filter_pass(skip):
            def piece(p, cnt_vec):
                pltpu.sync_copy(idx_hbm.at[pl.ds(p * _IP, _IP)], idxp)

                def scan(t, cv):
                    v = idxp[pl.ds(t * _L, _L)]
                    pos = p * _IP + t * _L + lanes
                    lc = v - col_lo
                    in_rng = jnp.logical_and(lc >= 0, lc < col_n)
                    in_rng = jnp.logical_or(
                        in_rng,
                        jnp.logical_and(has_tail, v >= _NFULL_TC * _TCW),
                    )
                    mi = in_rng.astype(jnp.int32)
                    slots = cv + plsc.cumsum(mi) - 1 - skip
                    okm = jnp.logical_and(
                        in_rng,
                        jnp.logical_and(slots >= 0, slots < _E),
                    )
                    slots_c = jnp.clip(slots, 0, _E - 1)
                    plsc.store_scatter(ent_lc, [slots_c], lc, mask=okm)
                    plsc.store_scatter(ent_pos, [slots_c], pos, mask=okm)
                    return cv + plsc.all_reduce_population_count(in_rng)

                return lax.fori_loop(0, _IP // _L, scan, cnt_vec)

            cnt_vec = lax.fori_loop(
                0, _B // _IP, piece, jnp.zeros((_L,), jnp.int32)
            )
            return lax.reduce_max(cnt_vec, (0,))

        def process_chunk(buf_sel, cb, width, n_groups, cnt_r):
            def grp(e, carry):
                lc = ent_lc[pl.ds(e * _L, _L)]
                pos = ent_pos[pl.ds(e * _L, _L)]
                valid = (e * _L + lanes) < cnt_r
                lcl = lc - cb
                m2 = jnp.logical_and(
                    valid, jnp.logical_and(lcl >= 0, lcl < width)
                )
                any_hit = lax.reduce_max(m2.astype(jnp.int32), (0,))

                @pl.when(any_hit > 0)
                def _():
                    lcc = jnp.where(m2, lcl, 0)
                    ss = jnp.zeros((_L,), jnp.float32)
                    vals = []
                    for r in range(_D):
                        rv = jnp.full((_L,), r, jnp.int32)
                        x = plsc.load_gather(chunks.at[buf_sel], [rv, lcc])
                        vals.append(x)
                        ss = ss + x * x
                    rinv = _fast_rsqrt(jnp.maximum(ss, jnp.float32(1e-24)))
                    fc = lax.reduce_max(firec[pl.ds(0, _L)], (0,))
                    slot = lax.rem(fc, _R)

                    @pl.when(fc >= _R)
                    def _():
                        pltpu.make_async_copy(
                            staging.at[0], out_p.at[posref.at[0]], scsem
                        ).wait()

                    for r in range(_D):
                        rv = jnp.full((_L,), r, jnp.int32)
                        plsc.store_scatter(
                            staging.at[slot], [lanes, rv], vals[r] * rinv
                        )
                    posref[slot, pl.ds(0, _L)] = jnp.where(m2, pos, _B)
                    pltpu.async_copy(
                        staging.at[slot], out_p.at[posref.at[slot]], scsem
                    )
                    firec[pl.ds(0, _L)] = jnp.full((_L,), fc + 1, jnp.int32)

                return carry

            lax.fori_loop(0, n_groups, grp, 0)

        def start_dma(k, b):
            for tr in range(_D // 8):
                pltpu.async_copy(
                    table_t.at[pl.ds(tr * 8, 8), pl.ds(col_lo + k * _CW, _CW)],
                    chunks.at[b, pl.ds(tr * 8, 8)],
                    sems.at[b],
                )

        def wait_dma(b):
            for tr in range(_D // 8):
                pltpu.make_async_copy(
                    table_t.at[pl.ds(tr * 8, 8), pl.ds(0, _CW)],
                    chunks.at[b, pl.ds(tr * 8, 8)],
                    sems.at[b],
                ).wait()

        n_full = my_ntc // 4
        extra_tc = my_ntc - n_full * 4

        def round_body(cnt_r):
            n_groups = (cnt_r + _L - 1) // _L

            @pl.when(n_full > 0)
            def _():
                start_dma(0, 0)

            def outer(t, carry):
                for b in range(2):
                    k = t * 2 + b

                    @pl.when(k < n_full)
                    def _():
                        wait_dma(b)

                        @pl.when(k + 1 < n_full)
                        def _():
                            start_dma(k + 1, 1 - b)

                        process_chunk(b, k * _CW, _CW, n_groups, cnt_r)

                return carry

            lax.fori_loop(0, (_TC_SHARE // 4 + 2) // 2, outer, 0)

            @pl.when(extra_tc > 0)
            def _():
                pltpu.async_copy(
                    table_t.at[:, pl.ds(col_lo + n_full * _CW, _TCW)],
                    chunks.at[0, :, pl.ds(0, _TCW)],
                    sems.at[0],
                )
                pltpu.make_async_copy(
                    table_t.at[:, pl.ds(0, _TCW)],
                    chunks.at[0, :, pl.ds(0, _TCW)],
                    sems.at[0],
                ).wait()
                process_chunk(0, n_full * _CW, _TCW, n_groups, cnt_r)

            @pl.when(has_tail)
            def _():
                pltpu.sync_copy(tail_hbm, chunks.at[0, :, pl.ds(0, _TCW)])
                process_chunk(0, tail_cb, _TAIL, n_groups, cnt_r)

        firec[pl.ds(0, _L)] = jnp.zeros((_L,), jnp.int32)
        cnt = filter_pass(0)
        round_body(jnp.minimum(cnt, _E))
        n_rounds = (cnt + _E - 1) // _E

        def extra_round(r, carry):
            filter_pass(r * _E)
            round_body(jnp.minimum(cnt - r * _E, _E))
            return carry

        lax.fori_loop(1, n_rounds, extra_round, 0)

        fc_end = lax.reduce_max(firec[pl.ds(0, _L)], (0,))

        def drain(_, carry):
            pltpu.make_async_copy(
                staging.at[0], out_p.at[posref.at[0]], scsem
            ).wait()
            return carry

        lax.fori_loop(0, jnp.minimum(fc_end, _R), drain, 0)

    return body


def kernel(action, table):
    idx = action.astype(jnp.int32)
    table_t = table.T
    tail = jnp.pad(table_t[:, _NFULL_TC * _TCW :], ((0, 0), (0, _TCW - _TAIL)))
    out_p = _make_kernel()(table_t, tail, idx)
    return out_p[:_B, :_D]


# speedup vs baseline: 16.0639x; 16.0436x over previous
import functools

import jax
import jax.numpy as jnp
from jax import lax
from jax.experimental import pallas as pl
from jax.experimental.pallas import tpu as pltpu
from jax.experimental.pallas import tpu_sc as plsc

_L = 16
_NC = 2
_NS = 16
_NW = _NC * _NS
_GCHUNK = 128


def _fast_rsqrt(x):
    i = plsc.bitcast(x, jnp.int32)
    i = jnp.int32(0x5F3759DF) - (i >> 1)
    y = plsc.bitcast(i, jnp.float32)
    xh = x * jnp.float32(0.5)
    for _ in range(3):
        y = y * (jnp.float32(1.5) - xh * y * y)
    return y


def _make_kernel(num_actions, batch, dim):
    per_w = batch // _NW
    n_chunks = per_w // _GCHUNK
    n_groups = per_w // _L
    mesh = plsc.VectorSubcoreMesh(core_axis_name="c", subcore_axis_name="s")

    @functools.partial(
        pl.kernel,
        out_type=jax.ShapeDtypeStruct((batch, dim), jnp.float32),
        mesh=mesh,
        scratch_types=[
            pltpu.VMEM((n_chunks, _GCHUNK), jnp.int32),
            pltpu.VMEM((per_w, dim), jnp.float32),
            pltpu.SemaphoreType.DMA((n_chunks,)),
        ],
        compiler_params=pltpu.CompilerParams(
            needs_layout_passes=False, use_tc_tiling_on_sc=False
        ),
    )
    def body(table_hbm, idx_hbm, out_hbm, idx_v, rows_v, sems):
        wid = lax.axis_index("s") * _NC + lax.axis_index("c")
        base = wid * per_w

        pltpu.sync_copy(idx_hbm.at[wid], idx_v)
        copies = [
            pltpu.async_copy(
                table_hbm.at[idx_v.at[j]],
                rows_v.at[pl.ds(j * _GCHUNK, _GCHUNK)],
                sems.at[j],
            )
            for j in range(n_chunks)
        ]
        for c in copies:
            c.wait()

        lanes = lax.iota(jnp.int32, _L)

        def group(g, carry):
            row_ids = g * _L + lanes
            ss = jnp.zeros((_L,), jnp.float32)
            cols = []
            for c in range(dim):
                cidx = jnp.full((_L,), c, jnp.int32)
                v = plsc.load_gather(rows_v, [row_ids, cidx])
                cols.append(v)
                ss = ss + v * v
            rinv = _fast_rsqrt(jnp.maximum(ss, jnp.float32(1e-24)))
            for c in range(dim):
                cidx = jnp.full((_L,), c, jnp.int32)
                plsc.store_scatter(rows_v, [row_ids, cidx], cols[c] * rinv)
            return carry

        lax.fori_loop(0, n_groups, group, 0)

        pltpu.sync_copy(rows_v, out_hbm.at[pl.ds(base, per_w)])

    return body


def kernel(action, table):
    num_actions, dim = table.shape
    (batch,) = action.shape
    idx = action.astype(jnp.int32).reshape(_NW, batch // (_NW * _GCHUNK), _GCHUNK)
    return _make_kernel(num_actions, batch, dim)(table, idx)
